# baseline (speedup 1.0000x reference)
import functools

import jax
import jax.numpy as jnp
from jax import lax
from jax.experimental import pallas as pl
from jax.experimental.pallas import tpu as pltpu
from jax.experimental.pallas import tpu_sc as plsc

_N_CLS = 1000
_D = 512
_P = 5
_C = 5
_T = 67
_M = _P + _C + _T

_PW = _P * _D
_CW = _C * _D
_TW = _T * _D
_MW = _M * _D

_info = plsc.get_sparse_core_info()
_NC = _info.num_cores
_NS = _info.num_subcores
_NW = _NC * _NS

_BASE = _N_CLS // _NW
_EXTRA = _N_CLS % _NW

_mesh = plsc.VectorSubcoreMesh(core_axis_name="c", subcore_axis_name="s")


@functools.partial(
    pl.kernel,
    mesh=_mesh,
    out_type=jax.ShapeDtypeStruct((_N_CLS, _MW), jnp.float32),
    scratch_types=[
        pltpu.VMEM((_MW,), jnp.float32),
    ],
)
def _prompt_concat(prefix_hbm, ctx_hbm, token_hbm, out_hbm, buf):
    core = lax.axis_index("c")
    sub = lax.axis_index("s")
    wid = sub * _NC + core
    cnt = _BASE + jnp.where(wid < _EXTRA, 1, 0)
    start = _BASE * wid + jnp.minimum(wid, _EXTRA)

    pltpu.sync_copy(prefix_hbm, buf.at[pl.ds(0, _PW)])

    def body(i, carry):
        c = start + i
        pltpu.sync_copy(ctx_hbm.at[c], buf.at[pl.ds(_PW, _CW)])
        pltpu.sync_copy(token_hbm.at[c], buf.at[pl.ds(_PW + _CW, _TW)])
        pltpu.sync_copy(buf, out_hbm.at[c])
        return carry

    lax.fori_loop(0, cnt, body, 0)


def kernel(prefix, ctx, token):
    out = _prompt_concat(
        prefix.reshape(_PW),
        ctx.reshape(_N_CLS, _CW),
        token.reshape(_N_CLS, _TW),
    )
    return out.reshape(_N_CLS, _M, _D)
